# Initial kernel scaffold; baseline (speedup 1.0000x reference)
import jax
import jax.numpy as jnp
from jax.experimental import pallas as pl


def kernel(node_attr, edge_index, edge_attr, edge_sh, fc_w1, fc_b1, fc_w2, fc_b2, bn_ws, bn_bs, bn_wv):
    raise NotImplementedError("write your pallas kernel here")


# speedup vs baseline: 2.8847x; 2.8847x over previous
import functools

import numpy as np
import jax
import jax.numpy as jnp
from jax import lax
from jax.experimental import pallas as pl
from jax.experimental.pallas import tpu as pltpu
from jax.experimental.pallas import tpu_sc as plsc

NS = 16
NV = 4
SH_DIM = 9
EF = 3 * NS
WN = NS * NS + NS * NV
EPS = 1e-5
N_NODES = 10000
N_EDGES = 320000
TP_W = 32

NCORES = 2
NSUB = 16
NW = NCORES * NSUB
PER_W = N_EDGES // NW
CHUNK = 80
NCHUNK = PER_W // CHUNK
ROWS_PER_TILE = N_NODES // NSUB

_R0 = np.repeat(np.eye(NS, dtype=np.float32), NS, axis=1)
_S0 = np.tile(np.eye(NS, dtype=np.float32), (NS, 1))
_R1 = np.repeat(np.eye(NS, dtype=np.float32), NV, axis=1)
_S1 = np.tile(np.eye(NV, dtype=np.float32), (NS, 1))
_E4 = np.repeat(np.eye(NV, dtype=np.float32), 3, axis=1)
_E3 = np.tile(np.eye(3, dtype=np.float32), (1, NV))
_M3 = np.kron(np.eye(NV, dtype=np.float32), np.ones((3, 3), np.float32) / 3.0)

_SC_MESH = dict(core_axis_name="c", subcore_axis_name="s",
                num_cores=NCORES, num_subcores=NSUB)


def _sc_gather_body(node_hbm, dst_hbm, out_hbm, idx_v, rows_v, sem):
    wid = lax.axis_index("s") * NCORES + lax.axis_index("c")
    base = wid * PER_W

    def body(i, carry):
        off = base + i * CHUNK
        pltpu.sync_copy(dst_hbm.at[pl.ds(off, CHUNK)], idx_v)
        pltpu.async_copy(node_hbm.at[idx_v], rows_v, sem).wait()
        pltpu.sync_copy(rows_v, out_hbm.at[pl.ds(off, CHUNK)])
        return carry

    lax.fori_loop(0, NCHUNK, body, 0)


def _sc_scatter_body(tp_hbm, src_hbm, z_hbm, out_hbm, idx_v, rows_v, copy_v, acc_sh):
    cid = lax.axis_index("c")
    sid = lax.axis_index("s")
    wid = sid * NCORES + cid
    base = wid * PER_W
    r0 = sid * ROWS_PER_TILE

    pltpu.sync_copy(z_hbm.at[pl.ds(r0, ROWS_PER_TILE)],
                    acc_sh.at[pl.ds(r0, ROWS_PER_TILE)])
    plsc.subcore_barrier()

    def body(i, carry):
        off = base + i * CHUNK
        pltpu.sync_copy(src_hbm.at[pl.ds(off, CHUNK)], idx_v)
        pltpu.sync_copy(tp_hbm.at[pl.ds(off, CHUNK)], rows_v)
        pltpu.sync_copy(rows_v, acc_sh.at[idx_v], add=True)
        return carry

    lax.fori_loop(0, NCHUNK, body, 0)
    plsc.subcore_barrier()

    pltpu.sync_copy(acc_sh.at[pl.ds(r0, ROWS_PER_TILE)], copy_v)
    pltpu.sync_copy(copy_v, out_hbm.at[cid, pl.ds(r0, ROWS_PER_TILE)])


@functools.cache
def _get_sc_gather():
    return pl.kernel(
        _sc_gather_body,
        out_type=jax.ShapeDtypeStruct((N_EDGES, NS), jnp.float32),
        mesh=plsc.VectorSubcoreMesh(**_SC_MESH),
        compiler_params=pltpu.CompilerParams(use_tc_tiling_on_sc=False),
        scratch_types=[
            pltpu.VMEM((CHUNK,), jnp.int32),
            pltpu.VMEM((CHUNK, NS), jnp.float32),
            pltpu.SemaphoreType.DMA,
        ],
    )


@functools.cache
def _get_sc_scatter():
    return pl.kernel(
        _sc_scatter_body,
        out_type=jax.ShapeDtypeStruct((NCORES, N_NODES, TP_W), jnp.float32),
        mesh=plsc.VectorSubcoreMesh(**_SC_MESH),
        compiler_params=pltpu.CompilerParams(use_tc_tiling_on_sc=False),
        scratch_types=[
            pltpu.VMEM((CHUNK,), jnp.int32),
            pltpu.VMEM((CHUNK, TP_W), jnp.float32),
            pltpu.VMEM((ROWS_PER_TILE, TP_W), jnp.float32),
            pltpu.VMEM_SHARED((N_NODES, TP_W), jnp.float32),
        ],
    )

B_EDGE = 1600
_GRID = N_EDGES // B_EDGE


def _tc_main_body(ea_ref, x_ref, sh_ref, w1_ref, b1_ref, w2_ref, b2_ref,
                  r0_ref, s0_ref, r1_ref, s1_ref, e4_ref, e3_ref, out_ref):
    f32 = jnp.float32
    h = jnp.dot(ea_ref[...], w1_ref[...], preferred_element_type=f32) + b1_ref[...]
    h = jnp.maximum(h, 0.0)
    w = jnp.dot(h, w2_ref[...], preferred_element_type=f32) + b2_ref[...]
    x = x_ref[...]
    p0 = jnp.dot(x, r0_ref[...], preferred_element_type=f32) * w[:, : NS * NS]
    pre0 = jnp.dot(p0, s0_ref[...], preferred_element_type=f32)
    p1 = jnp.dot(x, r1_ref[...], preferred_element_type=f32) * w[:, NS * NS :]
    pre1 = jnp.dot(p1, s1_ref[...], preferred_element_type=f32)
    sh = sh_ref[...]
    out0 = pre0 * sh[:, 0:1] * 0.25
    out1 = (jnp.dot(pre1, e4_ref[...], preferred_element_type=f32)
            * jnp.dot(sh[:, 1:4], e3_ref[...], preferred_element_type=f32)) * 0.25
    ones = jnp.ones((B_EDGE, 1), f32)
    zeros = jnp.zeros((B_EDGE, 3), f32)
    out_ref[...] = jnp.concatenate([out0, out1, ones, zeros], axis=1)


_tc_main = pl.pallas_call(
    _tc_main_body,
    grid=(_GRID,),
    in_specs=[
        pl.BlockSpec((B_EDGE, EF), lambda i: (i, 0)),
        pl.BlockSpec((B_EDGE, NS), lambda i: (i, 0)),
        pl.BlockSpec((B_EDGE, SH_DIM), lambda i: (i, 0)),
        pl.BlockSpec((EF, EF), lambda i: (0, 0)),
        pl.BlockSpec((1, EF), lambda i: (0, 0)),
        pl.BlockSpec((EF, WN), lambda i: (0, 0)),
        pl.BlockSpec((1, WN), lambda i: (0, 0)),
        pl.BlockSpec((NS, NS * NS), lambda i: (0, 0)),
        pl.BlockSpec((NS * NS, NS), lambda i: (0, 0)),
        pl.BlockSpec((NS, NS * NV), lambda i: (0, 0)),
        pl.BlockSpec((NS * NV, NV), lambda i: (0, 0)),
        pl.BlockSpec((NV, 12), lambda i: (0, 0)),
        pl.BlockSpec((3, 12), lambda i: (0, 0)),
    ],
    out_specs=pl.BlockSpec((B_EDGE, TP_W), lambda i: (i, 0)),
    out_shape=jax.ShapeDtypeStruct((N_EDGES, TP_W), jnp.float32),
)


def _tc_bn_body(pa_ref, pb_ref, ws_ref, bs_ref, wv_ref, m3_ref, out_ref):
    tot = pa_ref[...] + pb_ref[...]
    cnt = jnp.maximum(tot[:, 28:29], 1.0)
    mean_tp = tot[:, :28] / cnt
    s = mean_tp[:, :NS]
    v = mean_tp[:, NS:28]
    m = jnp.mean(s, axis=0, keepdims=True)
    var = jnp.mean((s - m) ** 2, axis=0, keepdims=True)
    s_out = (s - m) * lax.rsqrt(var + EPS) * ws_ref[...] + bs_ref[...]
    cm = jnp.mean(v * v, axis=0, keepdims=True)
    vn = jnp.dot(cm, m3_ref[...], preferred_element_type=jnp.float32)
    v_out = v * (wv_ref[...] * lax.rsqrt(vn + EPS))
    out_ref[...] = jnp.concatenate([s_out, v_out], axis=1)


_tc_bn = pl.pallas_call(
    _tc_bn_body,
    out_shape=jax.ShapeDtypeStruct((N_NODES, NS + 3 * NV), jnp.float32),
)


def kernel(node_attr, edge_index, edge_attr, edge_sh, fc_w1, fc_b1, fc_w2,
           fc_b2, bn_ws, bn_bs, bn_wv):
    x = _get_sc_gather()(node_attr, edge_index[1])
    tp = _tc_main(edge_attr, x, edge_sh, fc_w1, fc_b1.reshape(1, -1),
                  fc_w2, fc_b2.reshape(1, -1),
                  jnp.asarray(_R0), jnp.asarray(_S0), jnp.asarray(_R1),
                  jnp.asarray(_S1), jnp.asarray(_E4), jnp.asarray(_E3))
    zeros = jnp.zeros((N_NODES, TP_W), jnp.float32)
    parts = _get_sc_scatter()(tp, edge_index[0], zeros)
    out = _tc_bn(parts[0], parts[1], bn_ws.reshape(1, -1),
                 bn_bs.reshape(1, -1), jnp.repeat(bn_wv, 3).reshape(1, -1),
                 jnp.asarray(_M3))
    return out


# speedup vs baseline: 3.3389x; 1.1575x over previous
import functools

import numpy as np
import jax
import jax.numpy as jnp
from jax import lax
from jax.experimental import pallas as pl
from jax.experimental.pallas import tpu as pltpu
from jax.experimental.pallas import tpu_sc as plsc

NS = 16
NV = 4
SH_DIM = 9
EF = 3 * NS
WN = NS * NS + NS * NV
EPS = 1e-5
N_NODES = 10000
N_EDGES = 320000
TP_W = 32

NCORES = 2
NSUB = 16
NW = NCORES * NSUB
PER_W = N_EDGES // NW
CHUNK = 80
NCHUNK = PER_W // CHUNK
ROWS_PER_TILE = N_NODES // NSUB

_R0 = np.repeat(np.eye(NS, dtype=np.float32), NS, axis=1)
_S0 = np.tile(np.eye(NS, dtype=np.float32), (NS, 1))
_R1 = np.repeat(np.eye(NS, dtype=np.float32), NV, axis=1)
_S1 = np.tile(np.eye(NV, dtype=np.float32), (NS, 1))
_E4 = np.repeat(np.eye(NV, dtype=np.float32), 3, axis=1)
_E3 = np.tile(np.eye(3, dtype=np.float32), (1, NV))
_M3 = np.kron(np.eye(NV, dtype=np.float32), np.ones((3, 3), np.float32) / 3.0)

_SC_MESH = dict(core_axis_name="c", subcore_axis_name="s",
                num_cores=NCORES, num_subcores=NSUB)


G_SUPER = 25
G_ROWS = G_SUPER * CHUNK
G_NSUP = PER_W // G_ROWS


def _sc_gather_body(node_hbm, dst_hbm, out_hbm, idx_v, buf, gsem, ssem):
    wid = lax.axis_index("s") * NCORES + lax.axis_index("c")
    base = wid * PER_W
    pltpu.sync_copy(dst_hbm.at[pl.ds(base, PER_W)], idx_v)

    def super_body(s, carry):
        k = s % 2

        @pl.when(s >= 2)
        def _():
            pltpu.make_async_copy(
                buf.at[k], out_hbm.at[pl.ds(base, G_ROWS)], ssem).wait()

        descs = []
        for j in range(G_SUPER):
            off = s * G_ROWS + j * CHUNK
            descs.append(pltpu.async_copy(
                node_hbm.at[idx_v.at[pl.ds(off, CHUNK)]],
                buf.at[k, pl.ds(j * CHUNK, CHUNK)], gsem))
        for d in descs:
            d.wait()
        pltpu.async_copy(buf.at[k],
                         out_hbm.at[pl.ds(base + s * G_ROWS, G_ROWS)], ssem)
        return carry

    lax.fori_loop(0, G_NSUP, super_body, 0)
    for _ in range(min(2, G_NSUP)):
        pltpu.make_async_copy(
            buf.at[0], out_hbm.at[pl.ds(base, G_ROWS)], ssem).wait()


S_CHUNK = 125
S_NCHUNK = PER_W // S_CHUNK
S_SUPER = 8
S_ROWS = S_SUPER * S_CHUNK
S_NSUP = S_NCHUNK // S_SUPER


def _sc_scatter_body(tp_hbm, idx2_hbm, z_hbm, out_hbm, idx_v, buf, lsem,
                     scsem, acc_sh):
    cid = lax.axis_index("c")
    sid = lax.axis_index("s")
    wid = sid * NCORES + cid
    base = wid * PER_W
    r0 = sid * ROWS_PER_TILE

    pltpu.sync_copy(z_hbm.at[pl.ds(r0, ROWS_PER_TILE)],
                    acc_sh.at[pl.ds(r0, ROWS_PER_TILE)])
    pltpu.sync_copy(idx2_hbm.at[pl.ds(wid * S_NCHUNK, S_NCHUNK)], idx_v)
    plsc.subcore_barrier()
    pltpu.async_copy(tp_hbm.at[pl.ds(base, S_ROWS)], buf.at[0], lsem)

    def super_body(s, carry):
        k = s % 2
        pltpu.make_async_copy(
            tp_hbm.at[pl.ds(base, S_ROWS)], buf.at[k], lsem).wait()

        @pl.when(s + 1 < S_NSUP)
        def _():
            pltpu.async_copy(
                tp_hbm.at[pl.ds(base + (s + 1) * S_ROWS, S_ROWS)],
                buf.at[1 - k], lsem)

        descs = []
        for j in range(S_SUPER):
            descs.append(pltpu.async_copy(
                buf.at[k, pl.ds(j * S_CHUNK, S_CHUNK)],
                acc_sh.at[idx_v.at[s * S_SUPER + j]], scsem, add=True))
        for d in descs:
            d.wait()
        return carry

    lax.fori_loop(0, S_NSUP, super_body, 0)
    plsc.subcore_barrier()

    pltpu.sync_copy(acc_sh.at[pl.ds(r0, ROWS_PER_TILE)],
                    buf.at[0, pl.ds(0, ROWS_PER_TILE)])
    pltpu.sync_copy(buf.at[0, pl.ds(0, ROWS_PER_TILE)],
                    out_hbm.at[cid, pl.ds(r0, ROWS_PER_TILE)])


@functools.cache
def _get_sc_gather():
    return pl.kernel(
        _sc_gather_body,
        out_type=jax.ShapeDtypeStruct((N_EDGES, NS), jnp.float32),
        mesh=plsc.VectorSubcoreMesh(**_SC_MESH),
        compiler_params=pltpu.CompilerParams(use_tc_tiling_on_sc=False),
        scratch_types=[
            pltpu.VMEM((PER_W,), jnp.int32),
            pltpu.VMEM((2, G_ROWS, NS), jnp.float32),
            pltpu.SemaphoreType.DMA,
            pltpu.SemaphoreType.DMA,
        ],
    )


@functools.cache
def _get_sc_scatter():
    return pl.kernel(
        _sc_scatter_body,
        out_type=jax.ShapeDtypeStruct((NCORES, N_NODES, TP_W), jnp.float32),
        mesh=plsc.VectorSubcoreMesh(**_SC_MESH),
        compiler_params=pltpu.CompilerParams(use_tc_tiling_on_sc=False),
        scratch_types=[
            pltpu.VMEM((S_NCHUNK, S_CHUNK), jnp.int32),
            pltpu.VMEM((2, S_ROWS, TP_W), jnp.float32),
            pltpu.SemaphoreType.DMA,
            pltpu.SemaphoreType.DMA,
            pltpu.VMEM_SHARED((N_NODES, TP_W), jnp.float32),
        ],
    )

B_EDGE = 1600
_GRID = N_EDGES // B_EDGE


def _tc_main_body(ea_ref, x_ref, sh_ref, w1_ref, b1_ref, w2_ref, b2_ref,
                  r0_ref, s0_ref, r1_ref, s1_ref, e4_ref, e3_ref, out_ref):
    f32 = jnp.float32
    h = jnp.dot(ea_ref[...], w1_ref[...], preferred_element_type=f32) + b1_ref[...]
    h = jnp.maximum(h, 0.0)
    w = jnp.dot(h, w2_ref[...], preferred_element_type=f32) + b2_ref[...]
    x = x_ref[...]
    p0 = jnp.dot(x, r0_ref[...], preferred_element_type=f32) * w[:, : NS * NS]
    pre0 = jnp.dot(p0, s0_ref[...], preferred_element_type=f32)
    p1 = jnp.dot(x, r1_ref[...], preferred_element_type=f32) * w[:, NS * NS :]
    pre1 = jnp.dot(p1, s1_ref[...], preferred_element_type=f32)
    sh = sh_ref[...]
    out0 = pre0 * sh[:, 0:1] * 0.25
    out1 = (jnp.dot(pre1, e4_ref[...], preferred_element_type=f32)
            * jnp.dot(sh[:, 1:4], e3_ref[...], preferred_element_type=f32)) * 0.25
    ones = jnp.ones((B_EDGE, 1), f32)
    zeros = jnp.zeros((B_EDGE, 3), f32)
    out_ref[...] = jnp.concatenate([out0, out1, ones, zeros], axis=1)


_tc_main = pl.pallas_call(
    _tc_main_body,
    grid=(_GRID,),
    in_specs=[
        pl.BlockSpec((B_EDGE, EF), lambda i: (i, 0)),
        pl.BlockSpec((B_EDGE, NS), lambda i: (i, 0)),
        pl.BlockSpec((B_EDGE, SH_DIM), lambda i: (i, 0)),
        pl.BlockSpec((EF, EF), lambda i: (0, 0)),
        pl.BlockSpec((1, EF), lambda i: (0, 0)),
        pl.BlockSpec((EF, WN), lambda i: (0, 0)),
        pl.BlockSpec((1, WN), lambda i: (0, 0)),
        pl.BlockSpec((NS, NS * NS), lambda i: (0, 0)),
        pl.BlockSpec((NS * NS, NS), lambda i: (0, 0)),
        pl.BlockSpec((NS, NS * NV), lambda i: (0, 0)),
        pl.BlockSpec((NS * NV, NV), lambda i: (0, 0)),
        pl.BlockSpec((NV, 12), lambda i: (0, 0)),
        pl.BlockSpec((3, 12), lambda i: (0, 0)),
    ],
    out_specs=pl.BlockSpec((B_EDGE, TP_W), lambda i: (i, 0)),
    out_shape=jax.ShapeDtypeStruct((N_EDGES, TP_W), jnp.float32),
)


def _tc_bn_body(pa_ref, pb_ref, ws_ref, bs_ref, wv_ref, m3_ref, out_ref):
    tot = pa_ref[...] + pb_ref[...]
    cnt = jnp.maximum(tot[:, 28:29], 1.0)
    mean_tp = tot[:, :28] / cnt
    s = mean_tp[:, :NS]
    v = mean_tp[:, NS:28]
    m = jnp.mean(s, axis=0, keepdims=True)
    var = jnp.mean((s - m) ** 2, axis=0, keepdims=True)
    s_out = (s - m) * lax.rsqrt(var + EPS) * ws_ref[...] + bs_ref[...]
    cm = jnp.mean(v * v, axis=0, keepdims=True)
    vn = jnp.dot(cm, m3_ref[...], preferred_element_type=jnp.float32)
    v_out = v * (wv_ref[...] * lax.rsqrt(vn + EPS))
    out_ref[...] = jnp.concatenate([s_out, v_out], axis=1)


_tc_bn = pl.pallas_call(
    _tc_bn_body,
    out_shape=jax.ShapeDtypeStruct((N_NODES, NS + 3 * NV), jnp.float32),
)


def kernel(node_attr, edge_index, edge_attr, edge_sh, fc_w1, fc_b1, fc_w2,
           fc_b2, bn_ws, bn_bs, bn_wv):
    x = _get_sc_gather()(node_attr, edge_index[1])
    tp = _tc_main(edge_attr, x, edge_sh, fc_w1, fc_b1.reshape(1, -1),
                  fc_w2, fc_b2.reshape(1, -1),
                  jnp.asarray(_R0), jnp.asarray(_S0), jnp.asarray(_R1),
                  jnp.asarray(_S1), jnp.asarray(_E4), jnp.asarray(_E3))
    zeros = jnp.zeros((N_NODES, TP_W), jnp.float32)
    parts = _get_sc_scatter()(tp, edge_index[0].reshape(N_EDGES // S_CHUNK, S_CHUNK), zeros)
    out = _tc_bn(parts[0], parts[1], bn_ws.reshape(1, -1),
                 bn_bs.reshape(1, -1), jnp.repeat(bn_wv, 3).reshape(1, -1),
                 jnp.asarray(_M3))
    return out


# speedup vs baseline: 3.8989x; 1.1677x over previous
import functools

import numpy as np
import jax
import jax.numpy as jnp
from jax import lax
from jax.experimental import pallas as pl
from jax.experimental.pallas import tpu as pltpu
from jax.experimental.pallas import tpu_sc as plsc

NS = 16
NV = 4
SH_DIM = 9
EF = 3 * NS
WN = NS * NS + NS * NV
EPS = 1e-5
N_NODES = 10000
N_EDGES = 320000
TP_W = 32

NCORES = 2
NSUB = 16
NW = NCORES * NSUB
PER_W = N_EDGES // NW
CHUNK = 80
NCHUNK = PER_W // CHUNK
ROWS_PER_TILE = N_NODES // NSUB

_R0 = np.repeat(np.eye(NS, dtype=np.float32), NS, axis=1)
_S0 = np.tile(np.eye(NS, dtype=np.float32), (NS, 1))
_R1 = np.repeat(np.eye(NS, dtype=np.float32), NV, axis=1)
_S1 = np.tile(np.eye(NV, dtype=np.float32), (NS, 1))
_E4 = np.repeat(np.eye(NV, dtype=np.float32), 3, axis=1)
_E3 = np.tile(np.eye(3, dtype=np.float32), (1, NV))
_M3 = np.kron(np.eye(NV, dtype=np.float32), np.ones((3, 3), np.float32) / 3.0)

_SC_MESH = dict(core_axis_name="c", subcore_axis_name="s",
                num_cores=NCORES, num_subcores=NSUB)


G_SUPER = 25
G_ROWS = G_SUPER * CHUNK
G_NSUP = PER_W // G_ROWS


def _sc_gather_body(node_hbm, dst_hbm, out_hbm, idx_v, buf, gsem, ssem):
    wid = lax.axis_index("s") * NCORES + lax.axis_index("c")
    base = wid * PER_W
    pltpu.sync_copy(dst_hbm.at[pl.ds(base, PER_W)], idx_v)

    def super_body(s, carry):
        k = s % 2

        @pl.when(s >= 2)
        def _():
            pltpu.make_async_copy(
                buf.at[k], out_hbm.at[pl.ds(base, G_ROWS)], ssem).wait()

        descs = []
        for j in range(G_SUPER):
            off = s * G_ROWS + j * CHUNK
            descs.append(pltpu.async_copy(
                node_hbm.at[idx_v.at[pl.ds(off, CHUNK)]],
                buf.at[k, pl.ds(j * CHUNK, CHUNK)], gsem))
        for d in descs:
            d.wait()
        pltpu.async_copy(buf.at[k],
                         out_hbm.at[pl.ds(base + s * G_ROWS, G_ROWS)], ssem)
        return carry

    lax.fori_loop(0, G_NSUP, super_body, 0)
    for _ in range(min(2, G_NSUP)):
        pltpu.make_async_copy(
            buf.at[0], out_hbm.at[pl.ds(base, G_ROWS)], ssem).wait()


S_CHUNK = 125
S_NCHUNK = PER_W // S_CHUNK
S_SUPER = 8
S_ROWS = S_SUPER * S_CHUNK
S_NSUP = S_NCHUNK // S_SUPER


def _sc_scatter_body(tp_hbm, idx2_hbm, z_hbm, out_hbm, idx_v, buf, lsem,
                     scsem, acc_sh):
    cid = lax.axis_index("c")
    sid = lax.axis_index("s")
    wid = sid * NCORES + cid
    base = wid * PER_W
    r0 = sid * ROWS_PER_TILE

    pltpu.sync_copy(z_hbm.at[pl.ds(r0, ROWS_PER_TILE)],
                    acc_sh.at[pl.ds(r0, ROWS_PER_TILE)])
    pltpu.sync_copy(idx2_hbm.at[pl.ds(wid * S_NCHUNK, S_NCHUNK)], idx_v)
    plsc.subcore_barrier()
    pltpu.async_copy(tp_hbm.at[pl.ds(base, S_ROWS)], buf.at[0], lsem)

    def super_body(s, carry):
        k = s % 2
        pltpu.make_async_copy(
            tp_hbm.at[pl.ds(base, S_ROWS)], buf.at[k], lsem).wait()

        @pl.when(s + 1 < S_NSUP)
        def _():
            pltpu.async_copy(
                tp_hbm.at[pl.ds(base + (s + 1) * S_ROWS, S_ROWS)],
                buf.at[1 - k], lsem)

        descs = []
        for j in range(S_SUPER):
            descs.append(pltpu.async_copy(
                buf.at[k, pl.ds(j * S_CHUNK, S_CHUNK)],
                acc_sh.at[idx_v.at[s * S_SUPER + j]], scsem, add=True))
        for d in descs:
            d.wait()
        return carry

    lax.fori_loop(0, S_NSUP, super_body, 0)
    plsc.subcore_barrier()

    pltpu.sync_copy(acc_sh.at[pl.ds(r0, ROWS_PER_TILE)],
                    buf.at[0, pl.ds(0, ROWS_PER_TILE)])
    pltpu.sync_copy(buf.at[0, pl.ds(0, ROWS_PER_TILE)],
                    out_hbm.at[cid, pl.ds(r0, ROWS_PER_TILE)])


@functools.cache
def _get_sc_gather():
    return pl.kernel(
        _sc_gather_body,
        out_type=jax.ShapeDtypeStruct((N_EDGES, NS), jnp.float32),
        mesh=plsc.VectorSubcoreMesh(**_SC_MESH),
        compiler_params=pltpu.CompilerParams(use_tc_tiling_on_sc=False),
        scratch_types=[
            pltpu.VMEM((PER_W,), jnp.int32),
            pltpu.VMEM((2, G_ROWS, NS), jnp.float32),
            pltpu.SemaphoreType.DMA,
            pltpu.SemaphoreType.DMA,
        ],
    )


@functools.cache
def _get_sc_scatter():
    return pl.kernel(
        _sc_scatter_body,
        out_type=jax.ShapeDtypeStruct((NCORES, N_NODES, TP_W), jnp.float32),
        mesh=plsc.VectorSubcoreMesh(**_SC_MESH),
        compiler_params=pltpu.CompilerParams(use_tc_tiling_on_sc=False),
        scratch_types=[
            pltpu.VMEM((S_NCHUNK, S_CHUNK), jnp.int32),
            pltpu.VMEM((2, S_ROWS, TP_W), jnp.float32),
            pltpu.SemaphoreType.DMA,
            pltpu.SemaphoreType.DMA,
            pltpu.VMEM_SHARED((N_NODES, TP_W), jnp.float32),
        ],
    )

B_EDGE = 1280
_GRID = N_EDGES // B_EDGE


def _tc_main_body(ea_ref, x_ref, sh_ref, w1_ref, b1_ref, w2_ref, b2_ref,
                  r0_ref, s0_ref, r1_ref, s1_ref, e4_ref, e3_ref, out_ref):
    f32 = jnp.float32
    ea = ea_ref[...]
    h = jnp.maximum(
        jnp.dot(w1_ref[...], ea, preferred_element_type=f32) + b1_ref[...], 0.0)
    w = jnp.dot(w2_ref[...], h, preferred_element_type=f32) + b2_ref[...]
    x_t = x_ref[...]
    p0 = jnp.dot(r0_ref[...], x_t, preferred_element_type=f32) * w[: NS * NS]
    pre0 = jnp.dot(s0_ref[...], p0, preferred_element_type=f32)
    p1 = jnp.dot(r1_ref[...], x_t, preferred_element_type=f32) * w[NS * NS :]
    pre1 = jnp.dot(s1_ref[...], p1, preferred_element_type=f32)
    sh = sh_ref[...]
    out0 = pre0 * sh[0:1] * 0.25
    out1 = (jnp.dot(e4_ref[...], pre1, preferred_element_type=f32)
            * jnp.dot(e3_ref[...], sh[1:4], preferred_element_type=f32)) * 0.25
    ones = jnp.ones((1, B_EDGE), f32)
    zeros = jnp.zeros((3, B_EDGE), f32)
    out_ref[...] = jnp.concatenate([out0, out1, ones, zeros], axis=0)


_tc_main = pl.pallas_call(
    _tc_main_body,
    grid=(_GRID,),
    in_specs=[
        pl.BlockSpec((EF, B_EDGE), lambda i: (0, i)),
        pl.BlockSpec((NS, B_EDGE), lambda i: (0, i)),
        pl.BlockSpec((SH_DIM, B_EDGE), lambda i: (0, i)),
        pl.BlockSpec((EF, EF), lambda i: (0, 0)),
        pl.BlockSpec((EF, 1), lambda i: (0, 0)),
        pl.BlockSpec((WN, EF), lambda i: (0, 0)),
        pl.BlockSpec((WN, 1), lambda i: (0, 0)),
        pl.BlockSpec((NS * NS, NS), lambda i: (0, 0)),
        pl.BlockSpec((NS, NS * NS), lambda i: (0, 0)),
        pl.BlockSpec((NS * NV, NS), lambda i: (0, 0)),
        pl.BlockSpec((NV, NS * NV), lambda i: (0, 0)),
        pl.BlockSpec((12, NV), lambda i: (0, 0)),
        pl.BlockSpec((12, 3), lambda i: (0, 0)),
    ],
    out_specs=pl.BlockSpec((TP_W, B_EDGE), lambda i: (0, i)),
    out_shape=jax.ShapeDtypeStruct((TP_W, N_EDGES), jnp.float32),
)


def _tc_bn_body(pa_ref, pb_ref, ws_ref, bs_ref, wv_ref, m3_ref, out_ref):
    tot = pa_ref[...] + pb_ref[...]
    cnt = jnp.maximum(tot[:, 28:29], 1.0)
    mean_tp = tot[:, :28] / cnt
    s = mean_tp[:, :NS]
    v = mean_tp[:, NS:28]
    m = jnp.mean(s, axis=0, keepdims=True)
    var = jnp.mean((s - m) ** 2, axis=0, keepdims=True)
    s_out = (s - m) * lax.rsqrt(var + EPS) * ws_ref[...] + bs_ref[...]
    cm = jnp.mean(v * v, axis=0, keepdims=True)
    vn = jnp.dot(cm, m3_ref[...], preferred_element_type=jnp.float32)
    v_out = v * (wv_ref[...] * lax.rsqrt(vn + EPS))
    out_ref[...] = jnp.concatenate([s_out, v_out], axis=1)


_tc_bn = pl.pallas_call(
    _tc_bn_body,
    out_shape=jax.ShapeDtypeStruct((N_NODES, NS + 3 * NV), jnp.float32),
)


def kernel(node_attr, edge_index, edge_attr, edge_sh, fc_w1, fc_b1, fc_w2,
           fc_b2, bn_ws, bn_bs, bn_wv):
    x = _get_sc_gather()(node_attr, edge_index[1])
    tp2 = _tc_main(edge_attr.T, x.T, edge_sh.T,
                   fc_w1.T, fc_b1.reshape(-1, 1), fc_w2.T, fc_b2.reshape(-1, 1),
                   jnp.asarray(_R0.T), jnp.asarray(_S0.T), jnp.asarray(_R1.T),
                   jnp.asarray(_S1.T), jnp.asarray(_E4.T), jnp.asarray(_E3.T))
    tp = tp2.T
    zeros = jnp.zeros((N_NODES, TP_W), jnp.float32)
    parts = _get_sc_scatter()(tp, edge_index[0].reshape(N_EDGES // S_CHUNK, S_CHUNK), zeros)
    out = _tc_bn(parts[0], parts[1], bn_ws.reshape(1, -1),
                 bn_bs.reshape(1, -1), jnp.repeat(bn_wv, 3).reshape(1, -1),
                 jnp.asarray(_M3))
    return out


# speedup vs baseline: 4.7076x; 1.2074x over previous
import functools

import numpy as np
import jax
import jax.numpy as jnp
from jax import lax
from jax.experimental import pallas as pl
from jax.experimental.pallas import tpu as pltpu
from jax.experimental.pallas import tpu_sc as plsc

NS = 16
NV = 4
SH_DIM = 9
EF = 3 * NS
WN = NS * NS + NS * NV
EPS = 1e-5
N_NODES = 10000
N_EDGES = 320000
TP_W = 32

NCORES = 2
NSUB = 16
NW = NCORES * NSUB
PER_W = N_EDGES // NW
CHUNK = 80
NCHUNK = PER_W // CHUNK
ROWS_PER_TILE = N_NODES // NSUB

_R0 = np.repeat(np.eye(NS, dtype=np.float32), NS, axis=1)
_S0 = np.tile(np.eye(NS, dtype=np.float32), (NS, 1))
_R1 = np.repeat(np.eye(NS, dtype=np.float32), NV, axis=1)
_S1 = np.tile(np.eye(NV, dtype=np.float32), (NS, 1))
_E4 = np.repeat(np.eye(NV, dtype=np.float32), 3, axis=1)
_E3 = np.tile(np.eye(3, dtype=np.float32), (1, NV))
_M3 = np.kron(np.eye(NV, dtype=np.float32), np.ones((3, 3), np.float32) / 3.0)

_SC_MESH = dict(core_axis_name="c", subcore_axis_name="s",
                num_cores=NCORES, num_subcores=NSUB)


G_SUPER = 25
G_ROWS = G_SUPER * CHUNK
G_NSUP = PER_W // G_ROWS


def _sc_gather_body(node_hbm, dst_hbm, out_hbm, idx_v, buf, gsem, ssem):
    wid = lax.axis_index("s") * NCORES + lax.axis_index("c")
    base = wid * PER_W
    pltpu.sync_copy(dst_hbm.at[pl.ds(base, PER_W)], idx_v)

    def super_body(s, carry):
        k = s % 2

        @pl.when(s >= 2)
        def _():
            pltpu.make_async_copy(
                buf.at[k], out_hbm.at[pl.ds(base, G_ROWS)], ssem).wait()

        descs = []
        for j in range(G_SUPER):
            off = s * G_ROWS + j * CHUNK
            descs.append(pltpu.async_copy(
                node_hbm.at[idx_v.at[pl.ds(off, CHUNK)]],
                buf.at[k, pl.ds(j * CHUNK, CHUNK)], gsem))
        for d in descs:
            d.wait()
        pltpu.async_copy(buf.at[k],
                         out_hbm.at[pl.ds(base + s * G_ROWS, G_ROWS)], ssem)
        return carry

    lax.fori_loop(0, G_NSUP, super_body, 0)
    for _ in range(min(2, G_NSUP)):
        pltpu.make_async_copy(
            buf.at[0], out_hbm.at[pl.ds(base, G_ROWS)], ssem).wait()


S_CHUNK = 125
S_NCHUNK = PER_W // S_CHUNK
S_SUPER = 8
S_ROWS = S_SUPER * S_CHUNK
S_NSUP = S_NCHUNK // S_SUPER


def _sc_scatter_body(tp_hbm, idx2_hbm, z_hbm, out_hbm, idx_v, buf, lsem,
                     scsem, acc_sh):
    cid = lax.axis_index("c")
    sid = lax.axis_index("s")
    wid = sid * NCORES + cid
    base = wid * PER_W
    r0 = sid * ROWS_PER_TILE

    pltpu.sync_copy(z_hbm.at[pl.ds(r0, ROWS_PER_TILE)],
                    acc_sh.at[pl.ds(r0, ROWS_PER_TILE)])
    pltpu.sync_copy(idx2_hbm.at[pl.ds(wid * S_NCHUNK, S_NCHUNK)], idx_v)
    plsc.subcore_barrier()
    pltpu.async_copy(tp_hbm.at[pl.ds(base, S_ROWS)], buf.at[0], lsem)

    def super_body(s, carry):
        k = s % 2
        pltpu.make_async_copy(
            tp_hbm.at[pl.ds(base, S_ROWS)], buf.at[k], lsem).wait()

        @pl.when(s + 1 < S_NSUP)
        def _():
            pltpu.async_copy(
                tp_hbm.at[pl.ds(base + (s + 1) * S_ROWS, S_ROWS)],
                buf.at[1 - k], lsem)

        descs = []
        for j in range(S_SUPER):
            descs.append(pltpu.async_copy(
                buf.at[k, pl.ds(j * S_CHUNK, S_CHUNK)],
                acc_sh.at[idx_v.at[s * S_SUPER + j]], scsem, add=True))
        for d in descs:
            d.wait()
        return carry

    lax.fori_loop(0, S_NSUP, super_body, 0)
    plsc.subcore_barrier()

    pltpu.sync_copy(acc_sh.at[pl.ds(r0, ROWS_PER_TILE)],
                    buf.at[0, pl.ds(0, ROWS_PER_TILE)])
    pltpu.sync_copy(buf.at[0, pl.ds(0, ROWS_PER_TILE)],
                    out_hbm.at[cid, pl.ds(r0, ROWS_PER_TILE)])


@functools.cache
def _get_sc_gather():
    return pl.kernel(
        _sc_gather_body,
        out_type=jax.ShapeDtypeStruct((N_EDGES, NS), jnp.float32),
        mesh=plsc.VectorSubcoreMesh(**_SC_MESH),
        compiler_params=pltpu.CompilerParams(use_tc_tiling_on_sc=False),
        scratch_types=[
            pltpu.VMEM((PER_W,), jnp.int32),
            pltpu.VMEM((2, G_ROWS, NS), jnp.float32),
            pltpu.SemaphoreType.DMA,
            pltpu.SemaphoreType.DMA,
        ],
    )


@functools.cache
def _get_sc_scatter():
    return pl.kernel(
        _sc_scatter_body,
        out_type=jax.ShapeDtypeStruct((NCORES, N_NODES, TP_W), jnp.float32),
        mesh=plsc.VectorSubcoreMesh(**_SC_MESH),
        compiler_params=pltpu.CompilerParams(use_tc_tiling_on_sc=False),
        scratch_types=[
            pltpu.VMEM((S_NCHUNK, S_CHUNK), jnp.int32),
            pltpu.VMEM((2, S_ROWS, TP_W), jnp.float32),
            pltpu.SemaphoreType.DMA,
            pltpu.SemaphoreType.DMA,
            pltpu.VMEM_SHARED((N_NODES, TP_W), jnp.float32),
        ],
    )

B_EDGE = 12800
_GRID = N_EDGES // B_EDGE


def _tc_main_body(ea_ref, x_ref, sh_ref, w1_ref, w2_ref,
                  r0_ref, s0_ref, r1_ref, s1_ref, e4_ref, e3_ref, out_ref):
    f32 = jnp.float32
    ones = jnp.ones((1, B_EDGE), f32)
    ea = jnp.concatenate([ea_ref[...], ones], axis=0)
    h = jnp.maximum(jnp.dot(w1_ref[...], ea, preferred_element_type=f32), 0.0)
    h1 = jnp.concatenate([h, ones], axis=0)
    w = jnp.dot(w2_ref[...], h1, preferred_element_type=f32)
    x_t = x_ref[...]
    p0 = jnp.dot(r0_ref[...], x_t, preferred_element_type=f32) * w[: NS * NS]
    pre0 = jnp.dot(s0_ref[...], p0, preferred_element_type=f32)
    p1 = jnp.dot(r1_ref[...], x_t, preferred_element_type=f32) * w[NS * NS :]
    pre1 = jnp.dot(s1_ref[...], p1, preferred_element_type=f32)
    sh = sh_ref[...]
    out0 = pre0 * sh[0:1] * 0.25
    out1 = (jnp.dot(e4_ref[...], pre1, preferred_element_type=f32)
            * jnp.dot(e3_ref[...], sh[1:4], preferred_element_type=f32)) * 0.25
    zeros = jnp.zeros((3, B_EDGE), f32)
    out_ref[...] = jnp.concatenate([out0, out1, ones, zeros], axis=0)


_tc_main = pl.pallas_call(
    _tc_main_body,
    grid=(_GRID,),
    in_specs=[
        pl.BlockSpec((EF, B_EDGE), lambda i: (0, i)),
        pl.BlockSpec((NS, B_EDGE), lambda i: (0, i)),
        pl.BlockSpec((SH_DIM, B_EDGE), lambda i: (0, i)),
        pl.BlockSpec((EF, EF + 1), lambda i: (0, 0)),
        pl.BlockSpec((WN, EF + 1), lambda i: (0, 0)),
        pl.BlockSpec((NS * NS, NS), lambda i: (0, 0)),
        pl.BlockSpec((NS, NS * NS), lambda i: (0, 0)),
        pl.BlockSpec((NS * NV, NS), lambda i: (0, 0)),
        pl.BlockSpec((NV, NS * NV), lambda i: (0, 0)),
        pl.BlockSpec((12, NV), lambda i: (0, 0)),
        pl.BlockSpec((12, 3), lambda i: (0, 0)),
    ],
    out_specs=pl.BlockSpec((TP_W, B_EDGE), lambda i: (0, i)),
    out_shape=jax.ShapeDtypeStruct((TP_W, N_EDGES), jnp.float32),
)


def _tc_bn_body(pa_ref, pb_ref, ws_ref, bs_ref, wv_ref, m3_ref, out_ref):
    tot = pa_ref[...] + pb_ref[...]
    cnt = jnp.maximum(tot[:, 28:29], 1.0)
    mean_tp = tot[:, :28] / cnt
    s = mean_tp[:, :NS]
    v = mean_tp[:, NS:28]
    m = jnp.mean(s, axis=0, keepdims=True)
    var = jnp.mean((s - m) ** 2, axis=0, keepdims=True)
    s_out = (s - m) * lax.rsqrt(var + EPS) * ws_ref[...] + bs_ref[...]
    cm = jnp.mean(v * v, axis=0, keepdims=True)
    vn = jnp.dot(cm, m3_ref[...], preferred_element_type=jnp.float32)
    v_out = v * (wv_ref[...] * lax.rsqrt(vn + EPS))
    out_ref[...] = jnp.concatenate([s_out, v_out], axis=1)


_tc_bn = pl.pallas_call(
    _tc_bn_body,
    out_shape=jax.ShapeDtypeStruct((N_NODES, NS + 3 * NV), jnp.float32),
)


def kernel(node_attr, edge_index, edge_attr, edge_sh, fc_w1, fc_b1, fc_w2,
           fc_b2, bn_ws, bn_bs, bn_wv):
    x = _get_sc_gather()(node_attr, edge_index[1])
    w1a = jnp.concatenate([fc_w1.T, fc_b1.reshape(-1, 1)], axis=1)
    w2a = jnp.concatenate([fc_w2.T, fc_b2.reshape(-1, 1)], axis=1)
    tp2 = _tc_main(edge_attr.T, x.T, edge_sh.T, w1a, w2a,
                   jnp.asarray(_R0.T), jnp.asarray(_S0.T), jnp.asarray(_R1.T),
                   jnp.asarray(_S1.T), jnp.asarray(_E4.T), jnp.asarray(_E3.T))
    tp = tp2.T
    zeros = jnp.zeros((N_NODES, TP_W), jnp.float32)
    parts = _get_sc_scatter()(tp, edge_index[0].reshape(N_EDGES // S_CHUNK, S_CHUNK), zeros)
    out = _tc_bn(parts[0], parts[1], bn_ws.reshape(1, -1),
                 bn_bs.reshape(1, -1), jnp.repeat(bn_wv, 3).reshape(1, -1),
                 jnp.asarray(_M3))
    return out


# speedup vs baseline: 6.9618x; 1.4788x over previous
import functools

import numpy as np
import jax
import jax.numpy as jnp
from jax import lax
from jax.experimental import pallas as pl
from jax.experimental.pallas import tpu as pltpu
from jax.experimental.pallas import tpu_sc as plsc

NS = 16
NV = 4
SH_DIM = 9
EF = 3 * NS
WN = NS * NS + NS * NV
EPS = 1e-5
N_NODES = 10000
N_EDGES = 320000
TP_W = 32

NCORES = 2
NSUB = 16
NW = NCORES * NSUB
PER_W = N_EDGES // NW
CHUNK = 80
NCHUNK = PER_W // CHUNK
ROWS_PER_TILE = N_NODES // NSUB

_R0 = np.repeat(np.eye(NS, dtype=np.float32), NS, axis=1)
_S0 = np.tile(np.eye(NS, dtype=np.float32), (NS, 1))
_R1 = np.repeat(np.eye(NS, dtype=np.float32), NV, axis=1)
_S1 = np.tile(np.eye(NV, dtype=np.float32), (NS, 1))
_E4 = np.repeat(np.eye(NV, dtype=np.float32), 3, axis=1)
_E3 = np.tile(np.eye(3, dtype=np.float32), (1, NV))
_M3 = np.kron(np.eye(NV, dtype=np.float32), np.ones((3, 3), np.float32) / 3.0)

_SC_MESH = dict(core_axis_name="c", subcore_axis_name="s",
                num_cores=NCORES, num_subcores=NSUB)


G_SUPER = 25
G_ROWS = G_SUPER * CHUNK
G_NSUP = PER_W // G_ROWS


def _sc_gather_body(node_hbm, dst_hbm, out_hbm, idx_v, buf, gsem, ssem):
    wid = lax.axis_index("s") * NCORES + lax.axis_index("c")
    base = wid * PER_W
    pltpu.sync_copy(dst_hbm.at[pl.ds(base, PER_W)], idx_v)

    def super_body(s, carry):
        k = s % 2

        @pl.when(s >= 2)
        def _():
            pltpu.make_async_copy(
                buf.at[k], out_hbm.at[pl.ds(base, G_ROWS)], ssem).wait()

        descs = []
        for j in range(G_SUPER):
            off = s * G_ROWS + j * CHUNK
            descs.append(pltpu.async_copy(
                node_hbm.at[idx_v.at[pl.ds(off, CHUNK)]],
                buf.at[k, pl.ds(j * CHUNK, CHUNK)], gsem))
        for d in descs:
            d.wait()
        pltpu.async_copy(buf.at[k],
                         out_hbm.at[pl.ds(base + s * G_ROWS, G_ROWS)], ssem)
        return carry

    lax.fori_loop(0, G_NSUP, super_body, 0)
    for _ in range(min(2, G_NSUP)):
        pltpu.make_async_copy(
            buf.at[0], out_hbm.at[pl.ds(base, G_ROWS)], ssem).wait()


S_CHUNK = 125
S_NCHUNK = PER_W // S_CHUNK
S_SUPER = 8
S_ROWS = S_SUPER * S_CHUNK
S_NSUP = S_NCHUNK // S_SUPER


def _sc_scatter_body(tp_hbm, idx2_hbm, z_hbm, out_hbm, idx_v, buf, lsem,
                     scsem, acc_sh):
    cid = lax.axis_index("c")
    sid = lax.axis_index("s")
    wid = sid * NCORES + cid
    base = wid * PER_W
    r0 = sid * ROWS_PER_TILE

    pltpu.sync_copy(z_hbm.at[pl.ds(r0, ROWS_PER_TILE)],
                    acc_sh.at[pl.ds(r0, ROWS_PER_TILE)])
    pltpu.sync_copy(idx2_hbm.at[pl.ds(wid * S_NCHUNK, S_NCHUNK)], idx_v)
    plsc.subcore_barrier()
    pltpu.async_copy(tp_hbm.at[pl.ds(base, S_ROWS)], buf.at[0], lsem)

    def super_body(s, carry):
        k = s % 2
        pltpu.make_async_copy(
            tp_hbm.at[pl.ds(base, S_ROWS)], buf.at[k], lsem).wait()

        @pl.when(s + 1 < S_NSUP)
        def _():
            pltpu.async_copy(
                tp_hbm.at[pl.ds(base + (s + 1) * S_ROWS, S_ROWS)],
                buf.at[1 - k], lsem)

        descs = []
        for j in range(S_SUPER):
            descs.append(pltpu.async_copy(
                buf.at[k, pl.ds(j * S_CHUNK, S_CHUNK)],
                acc_sh.at[idx_v.at[s * S_SUPER + j]], scsem, add=True))
        for d in descs:
            d.wait()
        return carry

    lax.fori_loop(0, S_NSUP, super_body, 0)
    plsc.subcore_barrier()

    pltpu.sync_copy(acc_sh.at[pl.ds(r0, ROWS_PER_TILE)],
                    buf.at[0, pl.ds(0, ROWS_PER_TILE)])
    pltpu.sync_copy(buf.at[0, pl.ds(0, ROWS_PER_TILE)],
                    out_hbm.at[cid, pl.ds(r0, ROWS_PER_TILE)])


@functools.cache
def _get_sc_gather():
    return pl.kernel(
        _sc_gather_body,
        out_type=jax.ShapeDtypeStruct((N_EDGES, NS), jnp.float32),
        mesh=plsc.VectorSubcoreMesh(**_SC_MESH),
        compiler_params=pltpu.CompilerParams(use_tc_tiling_on_sc=False),
        scratch_types=[
            pltpu.VMEM((PER_W,), jnp.int32),
            pltpu.VMEM((2, G_ROWS, NS), jnp.float32),
            pltpu.SemaphoreType.DMA,
            pltpu.SemaphoreType.DMA,
        ],
    )


@functools.cache
def _get_sc_scatter():
    return pl.kernel(
        _sc_scatter_body,
        out_type=jax.ShapeDtypeStruct((NCORES, N_NODES, TP_W), jnp.float32),
        mesh=plsc.VectorSubcoreMesh(**_SC_MESH),
        compiler_params=pltpu.CompilerParams(use_tc_tiling_on_sc=False),
        scratch_types=[
            pltpu.VMEM((S_NCHUNK, S_CHUNK), jnp.int32),
            pltpu.VMEM((2, S_ROWS, TP_W), jnp.float32),
            pltpu.SemaphoreType.DMA,
            pltpu.SemaphoreType.DMA,
            pltpu.VMEM_SHARED((N_NODES, TP_W), jnp.float32),
        ],
    )

B_EDGE = 12800
_GRID = N_EDGES // B_EDGE


def _tc_main_body(ea_ref, x_ref, sh_ref, w1_ref, w2_ref,
                  r0_ref, s0_ref, r1_ref, s1_ref, e4_ref, e3_ref, out_ref):
    f32 = jnp.float32
    ones = jnp.ones((1, B_EDGE), f32)
    ea = jnp.concatenate([ea_ref[...], ones], axis=0)
    h = jnp.maximum(jnp.dot(w1_ref[...], ea, preferred_element_type=f32), 0.0)
    h1 = jnp.concatenate([h, ones], axis=0)
    w = jnp.dot(w2_ref[...], h1, preferred_element_type=f32)
    xp = x_ref[...]
    x_t = jnp.concatenate(
        [jnp.transpose(xp[:, NS * s:NS * (s + 1)]) for s in range(8)],
        axis=1)
    p0 = jnp.dot(r0_ref[...], x_t, preferred_element_type=f32) * w[: NS * NS]
    pre0 = jnp.dot(s0_ref[...], p0, preferred_element_type=f32)
    p1 = jnp.dot(r1_ref[...], x_t, preferred_element_type=f32) * w[NS * NS :]
    pre1 = jnp.dot(s1_ref[...], p1, preferred_element_type=f32)
    sh = sh_ref[...]
    out0 = pre0 * sh[0:1] * 0.25
    out1 = (jnp.dot(e4_ref[...], pre1, preferred_element_type=f32)
            * jnp.dot(e3_ref[...], sh[1:4], preferred_element_type=f32)) * 0.25
    zeros = jnp.zeros((3, B_EDGE), f32)
    tp_t = jnp.concatenate([out0, out1, ones, zeros], axis=0)
    q = B_EDGE // 4
    out_ref[...] = jnp.concatenate(
        [jnp.transpose(tp_t[:, q * e:q * (e + 1)]) for e in range(4)],
        axis=1)


_tc_main = pl.pallas_call(
    _tc_main_body,
    grid=(_GRID,),
    in_specs=[
        pl.BlockSpec((EF, B_EDGE), lambda i: (0, i)),
        pl.BlockSpec((B_EDGE // 8, 128), lambda i: (i, 0)),
        pl.BlockSpec((SH_DIM, B_EDGE), lambda i: (0, i)),
        pl.BlockSpec((EF, EF + 1), lambda i: (0, 0)),
        pl.BlockSpec((WN, EF + 1), lambda i: (0, 0)),
        pl.BlockSpec((NS * NS, NS), lambda i: (0, 0)),
        pl.BlockSpec((NS, NS * NS), lambda i: (0, 0)),
        pl.BlockSpec((NS * NV, NS), lambda i: (0, 0)),
        pl.BlockSpec((NV, NS * NV), lambda i: (0, 0)),
        pl.BlockSpec((12, NV), lambda i: (0, 0)),
        pl.BlockSpec((12, 3), lambda i: (0, 0)),
    ],
    out_specs=pl.BlockSpec((B_EDGE // 4, 128), lambda i: (i, 0)),
    out_shape=jax.ShapeDtypeStruct((N_EDGES // 4, 128), jnp.float32),
)


def _tc_bn_body(pa_ref, pb_ref, ws_ref, bs_ref, wv_ref, m3_ref, out_ref):
    tot = pa_ref[...] + pb_ref[...]
    cnt = jnp.maximum(tot[:, 28:29], 1.0)
    mean_tp = tot[:, :28] / cnt
    s = mean_tp[:, :NS]
    v = mean_tp[:, NS:28]
    m = jnp.mean(s, axis=0, keepdims=True)
    var = jnp.mean((s - m) ** 2, axis=0, keepdims=True)
    s_out = (s - m) * lax.rsqrt(var + EPS) * ws_ref[...] + bs_ref[...]
    cm = jnp.mean(v * v, axis=0, keepdims=True)
    vn = jnp.dot(cm, m3_ref[...], preferred_element_type=jnp.float32)
    v_out = v * (wv_ref[...] * lax.rsqrt(vn + EPS))
    out_ref[...] = jnp.concatenate([s_out, v_out], axis=1)


_tc_bn = pl.pallas_call(
    _tc_bn_body,
    out_shape=jax.ShapeDtypeStruct((N_NODES, NS + 3 * NV), jnp.float32),
)


def kernel(node_attr, edge_index, edge_attr, edge_sh, fc_w1, fc_b1, fc_w2,
           fc_b2, bn_ws, bn_bs, bn_wv):
    nb = N_EDGES // B_EDGE
    dst_sig = (edge_index[1].reshape(nb, 8, B_EDGE // 8)
               .transpose(0, 2, 1).reshape(N_EDGES))
    src_tau = (edge_index[0].reshape(nb, 4, B_EDGE // 4)
               .transpose(0, 2, 1).reshape(N_EDGES))
    x = _get_sc_gather()(node_attr, dst_sig)
    w1a = jnp.concatenate([fc_w1.T, fc_b1.reshape(-1, 1)], axis=1)
    w2a = jnp.concatenate([fc_w2.T, fc_b2.reshape(-1, 1)], axis=1)
    tp2 = _tc_main(edge_attr.T, x.reshape(N_EDGES // 8, 128), edge_sh.T,
                   w1a, w2a,
                   jnp.asarray(_R0.T), jnp.asarray(_S0.T), jnp.asarray(_R1.T),
                   jnp.asarray(_S1.T), jnp.asarray(_E4.T), jnp.asarray(_E3.T))
    tp = tp2.reshape(N_EDGES, TP_W)
    zeros = jnp.zeros((N_NODES, TP_W), jnp.float32)
    parts = _get_sc_scatter()(tp, src_tau.reshape(N_EDGES // S_CHUNK, S_CHUNK), zeros)
    out = _tc_bn(parts[0], parts[1], bn_ws.reshape(1, -1),
                 bn_bs.reshape(1, -1), jnp.repeat(bn_wv, 3).reshape(1, -1),
                 jnp.asarray(_M3))
    return out
